# baseline (speedup 1.0000x reference)
import functools

import jax
import jax.numpy as jnp
from jax import lax
from jax.experimental import pallas as pl
from jax.experimental.pallas import tpu as pltpu
from jax.experimental.pallas import tpu_sc as plsc

ROWS = 16384
COLS = 1000
LANES = 16

_INFO = plsc.get_sparse_core_info()
_NC = _INFO.num_cores
_NS = _INFO.num_subcores
_NW = _NC * _NS
ROWS_PER_W = ROWS // _NW
IDX_CHUNK = 128
N_CHUNKS = ROWS_PER_W // IDX_CHUNK


@functools.partial(
    pl.kernel,
    out_type=jax.ShapeDtypeStruct((ROWS,), jnp.float32),
    mesh=plsc.VectorSubcoreMesh(core_axis_name="c", subcore_axis_name="s"),
    scratch_types=[
        pltpu.VMEM((ROWS_PER_W,), jnp.int32),
        pltpu.VMEM((N_CHUNKS, IDX_CHUNK), jnp.int32),
        pltpu.VMEM((ROWS_PER_W,), jnp.float32),
        pltpu.SemaphoreType.DMA,
    ],
)
def _depth_gather(x_hbm, lab_hbm, out_hbm, lab_v, idx_v, val_v, sem):
    wid = lax.axis_index("s") * _NC + lax.axis_index("c")
    base = wid * ROWS_PER_W

    pltpu.sync_copy(lab_hbm.at[pl.ds(base, ROWS_PER_W)], lab_v)

    lane = lax.iota(jnp.int32, LANES)
    for j in range(ROWS_PER_W // LANES):
        r, c = divmod(j * LANES, IDX_CHUNK)
        labs = lab_v[pl.ds(j * LANES, LANES)]
        row = base + j * LANES + lane
        idx_v[r, pl.ds(c, LANES)] = row * COLS + labs

    copies = [
        pltpu.async_copy(
            x_hbm.at[idx_v.at[r]], val_v.at[pl.ds(r * IDX_CHUNK, IDX_CHUNK)], sem
        )
        for r in range(N_CHUNKS)
    ]
    for cp in copies:
        cp.wait()

    for j in range(ROWS_PER_W // LANES):
        v = val_v[pl.ds(j * LANES, LANES)]
        val_v[pl.ds(j * LANES, LANES)] = jnp.exp(jnp.abs(v) * 0.1) - 1.0

    pltpu.sync_copy(val_v, out_hbm.at[pl.ds(base, ROWS_PER_W)])


def kernel(x, labels):
    out = _depth_gather(x.reshape(-1), labels.astype(jnp.int32))
    return out[:, None]


# speedup vs baseline: 1.3105x; 1.3105x over previous
import functools

import jax
import jax.numpy as jnp
from jax import lax
from jax.experimental import pallas as pl
from jax.experimental.pallas import tpu as pltpu

ROWS = 16384
COLS = 1000
BLK_R = 256
GRID = ROWS // BLK_R


def _body(lab_ref, x_ref, out_ref):
    labs = lab_ref[0, 0, :]
    lab_col = jnp.reshape(labs, (BLK_R, 1))
    col = lax.broadcasted_iota(jnp.int32, (BLK_R, COLS), 1)
    v = jnp.sum(jnp.where(col == lab_col, x_ref[...], 0.0), axis=1)
    out_ref[0, 0, :] = jnp.exp(jnp.abs(v) * 0.1) - 1.0


@jax.jit
def kernel(x, labels):
    labs3 = labels.astype(jnp.int32).reshape(GRID, 1, BLK_R)
    out3 = pl.pallas_call(
        _body,
        grid=(GRID,),
        in_specs=[
            pl.BlockSpec((1, 1, BLK_R), lambda g: (g, 0, 0)),
            pl.BlockSpec((BLK_R, COLS), lambda g: (g, 0)),
        ],
        out_specs=pl.BlockSpec((1, 1, BLK_R), lambda g: (g, 0, 0)),
        out_shape=jax.ShapeDtypeStruct((GRID, 1, BLK_R), jnp.float32),
        compiler_params=pltpu.CompilerParams(
            dimension_semantics=("arbitrary",)
        ),
    )(labs3, x)
    return out3.reshape(ROWS, 1)
